# baseline (speedup 1.0000x reference)
import functools

import jax
import jax.numpy as jnp
from jax import lax
from jax.experimental import pallas as pl
from jax.experimental.pallas import tpu as pltpu
from jax.experimental.pallas import tpu_sc as plsc

U = 50000
D = 32
E = 800000
L_LAYERS = 3

NS = 16
CHUNK = 128
NCHUNK = 392
NSTORE = NCHUNK + 4
EPT = NSTORE * CHUNK
EPT_RAW = E // NS
GPAD = 0
SPAD = U
AROWS = 50048
ZPT = AROWS // NS
OPT = U // NS
DW = 8

_MESH = plsc.VectorSubcoreMesh(core_axis_name="c", subcore_axis_name="s")


@functools.partial(
    pl.kernel,
    out_type=(
        jax.ShapeDtypeStruct((U, D), jnp.float32),
        jax.ShapeDtypeStruct((U, D), jnp.float32),
    ),
    mesh=_MESH,
    scratch_types=[
        [pltpu.VMEM((CHUNK,), jnp.int32) for _ in range(4)],
        [pltpu.VMEM((CHUNK,), jnp.int32) for _ in range(4)],
        [pltpu.VMEM((CHUNK, D), jnp.float32) for _ in range(2)],
        pltpu.VMEM_SHARED((AROWS, D), jnp.float32),
        [pltpu.SemaphoreType.DMA for _ in range(4)],
        [pltpu.SemaphoreType.DMA for _ in range(2)],
    ],
    compiler_params=pltpu.CompilerParams(use_tc_tiling_on_sc=False),
)
def _propagate(hu, hi, eug, eus, eig, eis, zrows, outu, outi,
               cidx, ridx, gbuf, accum, isem, gsem):
    c = lax.axis_index("c")
    s = lax.axis_index("s")

    def run(tab, gih, sih, outh):
        base = s * EPT

        def fire_idx(j, b):
            pltpu.async_copy(gih.at[pl.ds(base + j * CHUNK, CHUNK)], cidx[b], isem[b])
            pltpu.async_copy(sih.at[pl.ds(base + j * CHUNK, CHUNK)], ridx[b], isem[b])

        def wait_idx(j, b):
            pltpu.make_async_copy(gih.at[pl.ds(base + j * CHUNK, CHUNK)], cidx[b], isem[b]).wait()
            pltpu.make_async_copy(sih.at[pl.ds(base + j * CHUNK, CHUNK)], ridx[b], isem[b]).wait()

        for b in range(4):
            fire_idx(b, b)
        pltpu.sync_copy(zrows, accum.at[pl.ds(s * ZPT, ZPT)])
        plsc.subcore_barrier()
        wait_idx(0, 0)
        pltpu.async_copy(tab.at[cidx[0]], gbuf[0], gsem[0])
        wait_idx(1, 1)
        pltpu.async_copy(tab.at[cidx[1]], gbuf[1], gsem[1])

        def outer(j0, _):
            j = j0 * 4
            for b in range(4):
                g = b % 2
                pltpu.make_async_copy(tab.at[cidx[b]], gbuf[g], gsem[g]).wait()
                pltpu.sync_copy(gbuf[g], accum.at[ridx[b]], add=True)
                fire_idx(j + b + 4, b)
                wait_idx(j + b + 2, (b + 2) % 4)
                pltpu.async_copy(tab.at[cidx[(b + 2) % 4]], gbuf[g], gsem[g])
            return 0
        lax.fori_loop(0, NCHUNK // 4, outer, 0)
        pltpu.make_async_copy(tab.at[cidx[0]], gbuf[0], gsem[0]).wait()
        pltpu.make_async_copy(tab.at[cidx[1]], gbuf[1], gsem[1]).wait()
        wait_idx(NCHUNK + 2, 2)
        wait_idx(NCHUNK + 3, 3)

        plsc.subcore_barrier()
        pltpu.sync_copy(accum.at[pl.ds(s * OPT, OPT)], outh.at[pl.ds(s * OPT, OPT)])

    pl.when(c == 0)(lambda: run(hi, eig, eus, outu))
    pl.when(c == 1)(lambda: run(hu, eug, eis, outi))


@functools.partial(
    pl.kernel,
    out_type=(
        jax.ShapeDtypeStruct((U, DW), jnp.float32),
        jax.ShapeDtypeStruct((U, DW), jnp.float32),
    ),
    mesh=_MESH,
    scratch_types=[
        [pltpu.VMEM((CHUNK,), jnp.int32) for _ in range(4)],
        pltpu.VMEM((CHUNK, DW), jnp.float32),
        pltpu.VMEM_SHARED((AROWS, DW), jnp.float32),
        [pltpu.SemaphoreType.DMA for _ in range(4)],
        [pltpu.SemaphoreType.DMA for _ in range(4)],
    ],
    compiler_params=pltpu.CompilerParams(use_tc_tiling_on_sc=False),
)
def _degree(eus, eis, ones_in, zrows, outu, outi, sidx, ones, dacc, isem, ssem):
    c = lax.axis_index("c")
    s = lax.axis_index("s")

    def run(sih, outh):
        base = s * EPT

        def fire_idx(j, b):
            pltpu.async_copy(sih.at[pl.ds(base + j * CHUNK, CHUNK)], sidx[b], isem[b])

        def wait_idx(j, b):
            pltpu.make_async_copy(sih.at[pl.ds(base + j * CHUNK, CHUNK)], sidx[b], isem[b]).wait()

        for b in range(4):
            fire_idx(b, b)
        ocp = pltpu.async_copy(ones_in, ones, isem[0])
        pltpu.sync_copy(zrows, dacc.at[pl.ds(s * ZPT, ZPT)])
        ocp.wait()
        plsc.subcore_barrier()

        def outer(j0, _):
            j = j0 * 4
            for b in range(4):
                wait_idx(j + b, b)
                pltpu.async_copy(ones, dacc.at[sidx[b]], ssem[b], add=True)
            for b in range(4):
                pltpu.make_async_copy(ones, dacc.at[sidx[b]], ssem[b]).wait()
                fire_idx(j + b + 4, b)
            return 0
        lax.fori_loop(0, NCHUNK // 4, outer, 0)
        for b in range(4):
            wait_idx(NCHUNK + b, b)

        plsc.subcore_barrier()
        pltpu.sync_copy(dacc.at[pl.ds(s * OPT, OPT)], outh.at[pl.ds(s * OPT, OPT)])

    pl.when(c == 0)(lambda: run(eus, outu))
    pl.when(c == 1)(lambda: run(eis, outi))


def _pad_edges(e, pad_val):
    r = e.reshape(NS, EPT_RAW)
    p = jnp.full((NS, EPT - EPT_RAW), pad_val, e.dtype)
    return jnp.concatenate([r, p], axis=1).reshape(-1)


def kernel(user_emb, item_emb, edge_user, edge_item):
    eu = edge_user.astype(jnp.int32)
    ei = edge_item.astype(jnp.int32)
    eug, eus = _pad_edges(eu, GPAD), _pad_edges(eu, SPAD)
    eig, eis = _pad_edges(ei, GPAD), _pad_edges(ei, SPAD)

    zrows = jnp.zeros((ZPT, D), jnp.float32)
    zrows_d = jnp.zeros((ZPT, DW), jnp.float32)
    ones_in = jnp.ones((CHUNK, DW), jnp.float32)

    degu, degi = _degree(eus, eis, ones_in, zrows_d)
    dinv_u = jnp.where(degu[:, :1] > 0, 1.0 / degu[:, :1], 0.0)
    dinv_i = jnp.where(degi[:, :1] > 0, 1.0 / degi[:, :1], 0.0)

    hu, hi = user_emb, item_emb
    acc_u, acc_i = hu, hi
    for _ in range(L_LAYERS):
        su, si = _propagate(hu, hi, eug, eus, eig, eis, zrows)
        hu = dinv_u * su
        hi = dinv_i * si
        acc_u = acc_u + hu
        acc_i = acc_i + hi

    scale = 1.0 / (L_LAYERS + 1)
    return jnp.concatenate([acc_u * scale, acc_i * scale], axis=0)


# speedup vs baseline: 1.1022x; 1.1022x over previous
import functools

import jax
import jax.numpy as jnp
from jax import lax
from jax.experimental import pallas as pl
from jax.experimental.pallas import tpu as pltpu
from jax.experimental.pallas import tpu_sc as plsc

U = 50000
D = 32
E = 800000
L_LAYERS = 3

NS = 16
CHUNK = 128
NCHUNK = 392
NSTORE = NCHUNK + 4
EPT = NSTORE * CHUNK
EPT_RAW = E // NS
GPAD = 0
SPAD = U
AROWS = 50048
ZPT = AROWS // NS
OPT = U // NS
NCH = OPT // CHUNK
NTAIL = OPT - NCH * CHUNK
DTW = 3152
DW = 8

_MESH = plsc.VectorSubcoreMesh(core_axis_name="c", subcore_axis_name="s")

_PROP_SCRATCH = [
    [pltpu.VMEM((CHUNK,), jnp.int32) for _ in range(4)],
    [pltpu.VMEM((CHUNK,), jnp.int32) for _ in range(4)],
    [pltpu.VMEM((CHUNK, D), jnp.float32) for _ in range(2)],
    pltpu.VMEM((CHUNK, D), jnp.float32),
    pltpu.VMEM((CHUNK, D), jnp.float32),
    pltpu.VMEM((CHUNK,), jnp.int32),
    pltpu.VMEM((DTW,), jnp.float32),
    pltpu.VMEM_SHARED((AROWS, D), jnp.float32),
    [pltpu.SemaphoreType.DMA for _ in range(4)],
    [pltpu.SemaphoreType.DMA for _ in range(2)],
    pltpu.SemaphoreType.DMA,
]


def _make_propagate(final):
    if final:
        out_type = jax.ShapeDtypeStruct((2 * U, D), jnp.float32)
    else:
        out_type = (
            jax.ShapeDtypeStruct((U, D), jnp.float32),
            jax.ShapeDtypeStruct((U, D), jnp.float32),
            jax.ShapeDtypeStruct((U, D), jnp.float32),
            jax.ShapeDtypeStruct((U, D), jnp.float32),
        )

    @functools.partial(
        pl.kernel,
        out_type=out_type,
        mesh=_MESH,
        scratch_types=_PROP_SCRATCH,
        compiler_params=pltpu.CompilerParams(use_tc_tiling_on_sc=False),
    )
    def prop(hu, hi, au, ai, du, di, eug, eus, eig, eis, zrows, *rest):
        if final:
            outs = rest[:1]
            rest = rest[1:]
        else:
            outs = rest[:4]
            rest = rest[4:]
        cidx, ridx, gbuf, abuf, cbuf, seqidx, dbuf, accum, isem, gsem, dsem = rest
        c = lax.axis_index("c")
        s = lax.axis_index("s")

        def run(tab, gih, sih, dinv, accin, write):
            base = s * EPT

            def fire_idx(j, b):
                pltpu.async_copy(gih.at[pl.ds(base + j * CHUNK, CHUNK)], cidx[b], isem[b])
                pltpu.async_copy(sih.at[pl.ds(base + j * CHUNK, CHUNK)], ridx[b], isem[b])

            def wait_idx(j, b):
                pltpu.make_async_copy(gih.at[pl.ds(base + j * CHUNK, CHUNK)], cidx[b], isem[b]).wait()
                pltpu.make_async_copy(sih.at[pl.ds(base + j * CHUNK, CHUNK)], ridx[b], isem[b]).wait()

            for b in range(4):
                fire_idx(b, b)
            dcp = pltpu.async_copy(dinv.at[s], dbuf, dsem)
            pltpu.sync_copy(zrows, accum.at[pl.ds(s * ZPT, ZPT)])
            plsc.subcore_barrier()
            wait_idx(0, 0)
            pltpu.async_copy(tab.at[cidx[0]], gbuf[0], gsem[0])
            wait_idx(1, 1)
            pltpu.async_copy(tab.at[cidx[1]], gbuf[1], gsem[1])

            def outer(j0, _):
                j = j0 * 4
                for b in range(4):
                    g = b % 2
                    pltpu.make_async_copy(tab.at[cidx[b]], gbuf[g], gsem[g]).wait()
                    pltpu.sync_copy(gbuf[g], accum.at[ridx[b]], add=True)
                    fire_idx(j + b + 4, b)
                    wait_idx(j + b + 2, (b + 2) % 4)
                    pltpu.async_copy(tab.at[cidx[(b + 2) % 4]], gbuf[g], gsem[g])
                return 0
            lax.fori_loop(0, NCHUNK // 4, outer, 0)
            pltpu.make_async_copy(tab.at[cidx[0]], gbuf[0], gsem[0]).wait()
            pltpu.make_async_copy(tab.at[cidx[1]], gbuf[1], gsem[1]).wait()
            wait_idx(NCHUNK + 2, 2)
            wait_idx(NCHUNK + 3, 3)

            plsc.subcore_barrier()
            dcp.wait()
            tile0 = s * OPT

            def chunk(c0, nr):
                row0 = tile0 + c0
                for t in range(0, CHUNK, 16):
                    seqidx[pl.ds(t, 16)] = jnp.minimum(
                        row0 + t + lax.iota(jnp.int32, 16), U - 1)
                ecp = pltpu.async_copy(accin.at[seqidx], cbuf, dsem)
                pltpu.sync_copy(accum.at[pl.ds(row0, nr)], abuf.at[pl.ds(0, nr)])
                ecp.wait()

                def body(r, _):
                    w = dbuf[pl.ds(c0 + r, 16)][0]
                    for h in range(0, D, 16):
                        v = abuf[r, pl.ds(h, 16)] * w
                        if final:
                            cbuf[r, pl.ds(h, 16)] = (cbuf[r, pl.ds(h, 16)] + v) * 0.25
                        else:
                            abuf[r, pl.ds(h, 16)] = v
                            cbuf[r, pl.ds(h, 16)] = cbuf[r, pl.ds(h, 16)] + v
                    return 0
                lax.fori_loop(0, nr, body, 0)
                write(row0, nr)

            def eloop(k, _):
                chunk(k * CHUNK, CHUNK)
                return 0
            lax.fori_loop(0, NCH, eloop, 0)
            chunk(NCH * CHUNK, NTAIL)

        if final:
            out = outs[0]

            def wr_u(row0, nr):
                pltpu.sync_copy(cbuf.at[pl.ds(0, nr)], out.at[pl.ds(row0, nr)])

            def wr_i(row0, nr):
                pltpu.sync_copy(cbuf.at[pl.ds(0, nr)], out.at[pl.ds(U + row0, nr)])
        else:
            outhu, outhi, outau, outai = outs

            def wr_u(row0, nr):
                pltpu.sync_copy(abuf.at[pl.ds(0, nr)], outhu.at[pl.ds(row0, nr)])
                pltpu.sync_copy(cbuf.at[pl.ds(0, nr)], outau.at[pl.ds(row0, nr)])

            def wr_i(row0, nr):
                pltpu.sync_copy(abuf.at[pl.ds(0, nr)], outhi.at[pl.ds(row0, nr)])
                pltpu.sync_copy(cbuf.at[pl.ds(0, nr)], outai.at[pl.ds(row0, nr)])

        pl.when(c == 0)(lambda: run(hi, eig, eus, du, au, wr_u))
        pl.when(c == 1)(lambda: run(hu, eug, eis, di, ai, wr_i))

    return prop


_prop_mid = _make_propagate(False)
_prop_last = _make_propagate(True)


@functools.partial(
    pl.kernel,
    out_type=(
        jax.ShapeDtypeStruct((U, DW), jnp.float32),
        jax.ShapeDtypeStruct((U, DW), jnp.float32),
    ),
    mesh=_MESH,
    scratch_types=[
        [pltpu.VMEM((CHUNK,), jnp.int32) for _ in range(4)],
        pltpu.VMEM((CHUNK, DW), jnp.float32),
        pltpu.VMEM_SHARED((AROWS, DW), jnp.float32),
        [pltpu.SemaphoreType.DMA for _ in range(4)],
        [pltpu.SemaphoreType.DMA for _ in range(4)],
    ],
    compiler_params=pltpu.CompilerParams(use_tc_tiling_on_sc=False),
)
def _degree(eus, eis, ones_in, zrows, outu, outi, sidx, ones, dacc, isem, ssem):
    c = lax.axis_index("c")
    s = lax.axis_index("s")

    def run(sih, outh):
        base = s * EPT

        def fire_idx(j, b):
            pltpu.async_copy(sih.at[pl.ds(base + j * CHUNK, CHUNK)], sidx[b], isem[b])

        def wait_idx(j, b):
            pltpu.make_async_copy(sih.at[pl.ds(base + j * CHUNK, CHUNK)], sidx[b], isem[b]).wait()

        for b in range(4):
            fire_idx(b, b)
        ocp = pltpu.async_copy(ones_in, ones, isem[0])
        pltpu.sync_copy(zrows, dacc.at[pl.ds(s * ZPT, ZPT)])
        ocp.wait()
        plsc.subcore_barrier()

        def outer(j0, _):
            j = j0 * 4
            for b in range(4):
                wait_idx(j + b, b)
                pltpu.async_copy(ones, dacc.at[sidx[b]], ssem[b], add=True)
            for b in range(4):
                pltpu.make_async_copy(ones, dacc.at[sidx[b]], ssem[b]).wait()
                fire_idx(j + b + 4, b)
            return 0
        lax.fori_loop(0, NCHUNK // 4, outer, 0)
        for b in range(4):
            wait_idx(NCHUNK + b, b)

        plsc.subcore_barrier()
        pltpu.sync_copy(dacc.at[pl.ds(s * OPT, OPT)], outh.at[pl.ds(s * OPT, OPT)])

    pl.when(c == 0)(lambda: run(eus, outu))
    pl.when(c == 1)(lambda: run(eis, outi))


def _pad_edges(e, pad_val):
    r = e.reshape(NS, EPT_RAW)
    p = jnp.full((NS, EPT - EPT_RAW), pad_val, e.dtype)
    return jnp.concatenate([r, p], axis=1).reshape(-1)


def kernel(user_emb, item_emb, edge_user, edge_item):
    eu = edge_user.astype(jnp.int32)
    ei = edge_item.astype(jnp.int32)
    eug, eus = _pad_edges(eu, GPAD), _pad_edges(eu, SPAD)
    eig, eis = _pad_edges(ei, GPAD), _pad_edges(ei, SPAD)

    zrows = jnp.zeros((ZPT, D), jnp.float32)
    zrows_d = jnp.zeros((ZPT, DW), jnp.float32)
    ones_in = jnp.ones((CHUNK, DW), jnp.float32)

    degu, degi = _degree(eus, eis, ones_in, zrows_d)
    duf = jnp.concatenate([jnp.where(degu[:, 0] > 0, 1.0 / degu[:, 0], 0.0),
                           jnp.zeros((DTW,), jnp.float32)])
    dif = jnp.concatenate([jnp.where(degi[:, 0] > 0, 1.0 / degi[:, 0], 0.0),
                           jnp.zeros((DTW,), jnp.float32)])
    du = jnp.stack([lax.dynamic_slice(duf, (i * OPT,), (DTW,)) for i in range(NS)])
    di = jnp.stack([lax.dynamic_slice(dif, (i * OPT,), (DTW,)) for i in range(NS)])

    hu, hi = user_emb, item_emb
    au, ai = user_emb, item_emb
    for _ in range(L_LAYERS - 1):
        hu, hi, au, ai = _prop_mid(hu, hi, au, ai, du, di,
                                   eug, eus, eig, eis, zrows)
    return _prop_last(hu, hi, au, ai, du, di, eug, eus, eig, eis, zrows)
